# baseline (speedup 1.0000x reference)
import functools

import jax
import jax.numpy as jnp
from jax import lax
from jax.experimental import pallas as pl
from jax.experimental.pallas import tpu as pltpu
from jax.experimental.pallas import tpu_sc as plsc

B = 16384
V = 1000000
D = 16
N_CAT = 10
D_IN = N_CAT * D + 3
H = 32

NC = 2
NS = 16
NW = NC * NS
TOT = B * N_CAT
PER_W = TOT // NW
ROWS_W = B // NW
CHUNK = 128
NCH = PER_W // CHUNK
NBUF = 4
L = 16


def _sc_gather(tabf, idx3):
    mesh = plsc.VectorSubcoreMesh(core_axis_name="c", subcore_axis_name="s")

    @functools.partial(
        pl.kernel,
        out_type=jax.ShapeDtypeStruct((NW, NCH, D, CHUNK), jnp.float32),
        mesh=mesh,
        scratch_types=[
            pltpu.VMEM((NCH, CHUNK), jnp.int32),
            pltpu.VMEM((2, D, CHUNK), jnp.int32),
            pltpu.VMEM((NBUF, D, CHUNK), jnp.float32),
            pltpu.SemaphoreType.DMA((NBUF,)),
            pltpu.SemaphoreType.DMA((NBUF,)),
        ],
        compiler_params=pltpu.CompilerParams(
            use_tc_tiling_on_sc=False, needs_layout_passes=False),
    )
    def k(tab_hbm, idx_hbm, out_hbm, idx_v, idxe_v, raw_v, esems, wsems):
        wid = lax.axis_index("s") * NC + lax.axis_index("c")
        pltpu.sync_copy(idx_hbm.at[wid], idx_v)

        iota_v = lax.iota(jnp.int32, L) * V

        def build(j):
            jb = lax.rem(j, 2)
            jv = jnp.full((L,), j, jnp.int32)
            for m in range(D):
                for g in range(CHUNK // L):
                    base = plsc.load_gather(
                        idx_v, [jv, jnp.full((L,), m * 8 + g, jnp.int32)])
                    idxe_v[jb, m, pl.ds(g * L, L)] = base + iota_v

        def fire_e(j):
            jb = lax.rem(j, 2)
            rb = lax.rem(j, NBUF)
            for m in range(D):
                pltpu.make_async_copy(
                    tab_hbm.at[idxe_v.at[jb, m]], raw_v.at[rb, m],
                    esems.at[rb],
                ).start()

        def wait_e(j):
            rb = lax.rem(j, NBUF)
            pltpu.make_async_copy(
                tab_hbm.at[idxe_v.at[0, 0]], raw_v.at[rb], esems.at[rb]
            ).wait()

        def fire_w(j):
            rb = lax.rem(j, NBUF)
            pltpu.make_async_copy(
                raw_v.at[rb], out_hbm.at[wid, j], wsems.at[rb],
            ).start()

        def wait_w(j):
            rb = lax.rem(j, NBUF)
            pltpu.make_async_copy(
                raw_v.at[rb], out_hbm.at[wid, 0], wsems.at[rb]
            ).wait()

        build(0)
        fire_e(0)
        build(1)
        fire_e(1)

        def step(j, carry):
            wait_e(j)
            fire_w(j)

            @pl.when(j + 2 < NCH)
            def _():
                @pl.when(j >= 2)
                def _():
                    wait_w(j + 2)
                build(j + 2)
                fire_e(j + 2)

            return carry

        lax.fori_loop(0, NCH, step, 0)
        for t in range(NBUF):
            wait_w(t)

    return k(tabf, idx3)


def _mlp_body(emb_ref, cont_ref, w10, b10, wo0, bo0, w11, b11, wo1, bo1,
              wout, bout, out_ref):
    x = jnp.concatenate([emb_ref[...], cont_ref[...]], axis=1)
    for (w1, b1, wo, bo) in ((w10, b10, wo0, bo0), (w11, b11, wo1, bo1)):
        h = jnp.maximum(
            jnp.dot(x, w1[...], preferred_element_type=jnp.float32) + b1[...],
            0.0)
        o = jnp.dot(h, wo[...], preferred_element_type=jnp.float32) + bo[...]
        x = jnp.maximum(o + x, 0.0)
    z = jnp.dot(x, wout[...], preferred_element_type=jnp.float32) + bout[...]
    out_ref[...] = jax.nn.sigmoid(z)


def _mlp(emb_flat, cont, w10, b10, wo0, bo0, w11, b11, wo1, bo1, wout, bout,
         blk=2048):
    grid = (B // blk,)
    full = lambda shape: pl.BlockSpec(shape, lambda i: (0, 0))
    return pl.pallas_call(
        _mlp_body,
        grid=grid,
        in_specs=[
            pl.BlockSpec((blk, N_CAT * D), lambda i: (i, 0)),
            pl.BlockSpec((blk, 3), lambda i: (i, 0)),
            full((D_IN, H)), full((1, H)), full((H, D_IN)), full((1, D_IN)),
            full((D_IN, H)), full((1, H)), full((H, D_IN)), full((1, D_IN)),
            full((D_IN, 1)), full((1, 1)),
        ],
        out_specs=pl.BlockSpec((blk, 1), lambda i: (i, 0)),
        out_shape=jax.ShapeDtypeStruct((B, 1), jnp.float32),
    )(emb_flat, cont, w10, b10, wo0, bo0, w11, b11, wo1, bo1, wout, bout)


def kernel(uid, iid, utag1, utag2, utag3, utag4, itag1, itag2, itag3, itag4,
           itag4_origin, itag4_square, itag4_cube,
           embed, W1_0, b1_0, Wo_0, bo_0, W1_1, b1_1, Wo_1, bo_1, Wout, bout):
    x_cate = jnp.concatenate(
        [uid, iid, utag1, utag2, utag3, utag4, itag1, itag2, itag3, itag4],
        axis=1)
    idx = x_cate.reshape(NW, NCH, CHUNK)
    tabf = embed.T.reshape(-1)
    rows = _sc_gather(tabf, idx)
    emb_flat = rows.reshape(B, N_CAT * D)
    cont = jnp.concatenate([itag4_origin, itag4_square, itag4_cube], axis=1)
    return _mlp(emb_flat, cont,
                W1_0, b1_0.reshape(1, H), Wo_0, bo_0.reshape(1, D_IN),
                W1_1, b1_1.reshape(1, H), Wo_1, bo_1.reshape(1, D_IN),
                Wout, bout.reshape(1, 1))


# speedup vs baseline: 2.7959x; 2.7959x over previous
import functools

import jax
import jax.numpy as jnp
from jax import lax
from jax.experimental import pallas as pl
from jax.experimental.pallas import tpu as pltpu
from jax.experimental.pallas import tpu_sc as plsc

B = 16384
V = 1000000
D = 16
N_CAT = 10
D_IN = N_CAT * D + 3
H = 32

NC = 2
NS = 16
NW = NC * NS
TOT = B * N_CAT
PER_W = TOT // NW
CHUNK = 128
NCH = PER_W // CHUNK


def _sc_gather(table, idx):
    mesh = plsc.VectorSubcoreMesh(core_axis_name="c", subcore_axis_name="s")

    @functools.partial(
        pl.kernel,
        out_type=jax.ShapeDtypeStruct((NW, NCH, CHUNK, D), jnp.float32),
        mesh=mesh,
        scratch_types=[
            pltpu.VMEM((NCH, CHUNK), jnp.int32),
            pltpu.VMEM((NCH, CHUNK, D), jnp.float32),
            pltpu.SemaphoreType.DMA,
        ],
        compiler_params=pltpu.CompilerParams(use_tc_tiling_on_sc=False),
    )
    def k(table_hbm, idx_hbm, out_hbm, idx_v, rows_v, sem):
        wid = lax.axis_index("s") * NC + lax.axis_index("c")
        pltpu.sync_copy(idx_hbm.at[wid], idx_v)

        def start(j, carry):
            pltpu.make_async_copy(
                table_hbm.at[idx_v.at[j]], rows_v.at[j], sem
            ).start()
            return carry

        lax.fori_loop(0, NCH, start, 0)
        pltpu.make_async_copy(out_hbm.at[wid], rows_v, sem).wait()
        pltpu.sync_copy(rows_v, out_hbm.at[wid])

    return k(table, idx)


def _mlp_body(emb_ref, cont_ref, w10, b10, wo0, bo0, w11, b11, wo1, bo1,
              wout, bout, out_ref):
    x = jnp.concatenate([emb_ref[...], cont_ref[...]], axis=1)
    for (w1, b1, wo, bo) in ((w10, b10, wo0, bo0), (w11, b11, wo1, bo1)):
        h = jnp.maximum(
            jnp.dot(x, w1[...], preferred_element_type=jnp.float32) + b1[...],
            0.0)
        o = jnp.dot(h, wo[...], preferred_element_type=jnp.float32) + bo[...]
        x = jnp.maximum(o + x, 0.0)
    z = jnp.dot(x, wout[...], preferred_element_type=jnp.float32) + bout[...]
    out_ref[...] = jax.nn.sigmoid(z)


def _mlp(emb_flat, cont, w10, b10, wo0, bo0, w11, b11, wo1, bo1, wout, bout,
         blk=2048):
    grid = (B // blk,)
    full = lambda shape: pl.BlockSpec(shape, lambda i: (0, 0))
    return pl.pallas_call(
        _mlp_body,
        grid=grid,
        in_specs=[
            pl.BlockSpec((blk, N_CAT * D), lambda i: (i, 0)),
            pl.BlockSpec((blk, 3), lambda i: (i, 0)),
            full((D_IN, H)), full((1, H)), full((H, D_IN)), full((1, D_IN)),
            full((D_IN, H)), full((1, H)), full((H, D_IN)), full((1, D_IN)),
            full((D_IN, 1)), full((1, 1)),
        ],
        out_specs=pl.BlockSpec((blk, 1), lambda i: (i, 0)),
        out_shape=jax.ShapeDtypeStruct((B, 1), jnp.float32),
    )(emb_flat, cont, w10, b10, wo0, bo0, w11, b11, wo1, bo1, wout, bout)


def kernel(uid, iid, utag1, utag2, utag3, utag4, itag1, itag2, itag3, itag4,
           itag4_origin, itag4_square, itag4_cube,
           embed, W1_0, b1_0, Wo_0, bo_0, W1_1, b1_1, Wo_1, bo_1, Wout, bout):
    x_cate = jnp.concatenate(
        [uid, iid, utag1, utag2, utag3, utag4, itag1, itag2, itag3, itag4],
        axis=1)
    idx = x_cate.reshape(NW, NCH, CHUNK)
    rows = _sc_gather(embed, idx)
    emb_flat = rows.reshape(B, N_CAT * D)
    cont = jnp.concatenate([itag4_origin, itag4_square, itag4_cube], axis=1)
    return _mlp(emb_flat, cont,
                W1_0, b1_0.reshape(1, H), Wo_0, bo_0.reshape(1, D_IN),
                W1_1, b1_1.reshape(1, H), Wo_1, bo_1.reshape(1, D_IN),
                Wout, bout.reshape(1, 1))
